# Initial kernel scaffold; baseline (speedup 1.0000x reference)
import jax
import jax.numpy as jnp
from jax.experimental import pallas as pl


def kernel(x, gate_w, W1, W2, W3, S1, S2, S3):
    raise NotImplementedError("write your pallas kernel here")


# speedup vs baseline: 1.3191x; 1.3191x over previous
import functools

import jax
import jax.numpy as jnp
from jax import lax
from jax.experimental import pallas as pl
from jax.experimental.pallas import tpu as pltpu
from jax.experimental.pallas import tpu_sc as plsc

T = 2048
D = 1024
DFF = 4096
E = 8
CAP = 307
CAP_PAD = 384
NROWS = E * CAP_PAD
NSH = 2
DSH = DFF // 2

_HI = jax.lax.Precision.HIGHEST


def _router_body(x_ref, gw_ref, out_ref):
    logits = jnp.dot(x_ref[...], gw_ref[...], preferred_element_type=jnp.float32,
                     precision=_HI)
    mx = jnp.max(logits, axis=1, keepdims=True)
    iot = lax.broadcasted_iota(jnp.int32, (T, E), 1)
    pick = jnp.where(logits == mx, iot, E)
    out_ref[...] = jnp.min(pick, axis=1, keepdims=True)


def _router(xr, gate_w):
    return pl.pallas_call(
        _router_body,
        out_shape=jax.ShapeDtypeStruct((T, 1), jnp.int32),
    )(xr, gate_w)


_NC = 2
_NS = 16
_NW = _NC * _NS
_GROWS = NROWS // _NW


def _routing_body(eidx_hbm, x_hbm, gathered_hbm, slot_hbm, kept_hbm, counts_hbm,
                  eidx_v, ids_v, slot_v, kept_v, cnt_v, ids_sh, idx_v, rows_v, sem):
    cid = lax.axis_index("c")
    sid = lax.axis_index("s")
    wid = sid * _NC + cid

    @pl.when(sid == 0)
    def _routing():
        pltpu.sync_copy(eidx_hbm, eidx_v)
        zero16 = jnp.zeros((16,), jnp.int32)
        for i in range(NROWS // 16):
            ids_v[pl.ds(16 * i, 16)] = zero16
        lanes = lax.iota(jnp.int32, 16)

        def body(i, carry):
            v = eidx_v[pl.ds(i * 16, 16)]
            tok = i * 16 + lanes
            slotv = jnp.zeros((16,), jnp.int32)
            keepv = jnp.zeros((16,), jnp.bool_)
            new_carry = []
            for e in range(E):
                m = v == e
                incl = plsc.cumsum(m.astype(jnp.int32))
                rank = carry[e] + incl - 1
                cnt = plsc.all_reduce_population_count(m)
                k = m & (rank < CAP)
                slotv = jnp.where(k, e * CAP_PAD + rank, slotv)
                keepv = keepv | k
                new_carry.append(carry[e] + cnt)
            slot_v[pl.ds(i * 16, 16)] = slotv
            kept_v[pl.ds(i * 16, 16)] = jnp.where(keepv, 1.0, 0.0)
            plsc.store_scatter(ids_v, [slotv], tok, mask=keepv)
            return tuple(new_carry)

        init = tuple(jnp.zeros((16,), jnp.int32) for _ in range(E))
        carry = lax.fori_loop(0, T // 16, body, init)
        cvec = jnp.zeros((16,), jnp.int32)
        for e in range(E):
            cvec = jnp.where(lanes == e, carry[e], cvec)
        cnt_v[...] = cvec
        pltpu.sync_copy(ids_v, ids_sh)

        @pl.when(cid == 0)
        def _hbm_outs():
            pltpu.sync_copy(cnt_v, counts_hbm)
            pltpu.sync_copy(slot_v, slot_hbm)
            pltpu.sync_copy(kept_v, kept_hbm)

    plsc.subcore_barrier()
    base = wid * _GROWS
    pltpu.sync_copy(ids_sh.at[pl.ds(base, _GROWS)], idx_v)
    pltpu.async_copy(x_hbm.at[idx_v], rows_v, sem).wait()
    pltpu.sync_copy(rows_v, gathered_hbm.at[pl.ds(base, _GROWS)])


def _routing_gather(eidx, xr):
    mesh = plsc.VectorSubcoreMesh(core_axis_name="c", subcore_axis_name="s",
                                  num_cores=_NC, num_subcores=_NS)
    return pl.kernel(
        _routing_body,
        out_type=[
            jax.ShapeDtypeStruct((NROWS, D), jnp.float32),
            jax.ShapeDtypeStruct((T,), jnp.int32),
            jax.ShapeDtypeStruct((T,), jnp.float32),
            jax.ShapeDtypeStruct((16,), jnp.int32),
        ],
        mesh=mesh,
        scratch_types=[
            pltpu.VMEM((T,), jnp.int32),
            pltpu.VMEM((NROWS,), jnp.int32),
            pltpu.VMEM((T,), jnp.int32),
            pltpu.VMEM((T,), jnp.float32),
            pltpu.VMEM((16,), jnp.int32),
            pltpu.VMEM_SHARED((NROWS,), jnp.int32),
            pltpu.VMEM((_GROWS,), jnp.int32),
            pltpu.VMEM((_GROWS, D), jnp.float32),
            pltpu.SemaphoreType.DMA,
        ],
        compiler_params=pltpu.CompilerParams(needs_layout_passes=False),
    )(eidx, xr)


_KC = 512
_NKC = DFF // _KC


def _ffn_body(g_ref, w1_ref, w2_ref, w3_ref, out_ref):
    g = g_ref[...]
    h1 = jnp.dot(g, w1_ref[0], preferred_element_type=jnp.float32)
    h2 = jnp.dot(g, w2_ref[0], preferred_element_type=jnp.float32)
    h = h1 * (1.0 / (1.0 + jnp.exp(-h1))) * h2
    o = jnp.dot(h, w3_ref[0], preferred_element_type=jnp.float32)

    @pl.when(pl.program_id(1) == 0)
    def _init():
        out_ref[...] = o

    @pl.when(pl.program_id(1) != 0)
    def _acc():
        out_ref[...] += o


def _expert_ffn(gathered, W1, W2, W3):
    return pl.pallas_call(
        _ffn_body,
        grid=(E, _NKC),
        in_specs=[
            pl.BlockSpec((CAP_PAD, D), lambda e, k: (e, 0)),
            pl.BlockSpec((1, D, _KC), lambda e, k: (e, 0, k)),
            pl.BlockSpec((1, D, _KC), lambda e, k: (e, 0, k)),
            pl.BlockSpec((1, _KC, D), lambda e, k: (e, k, 0)),
        ],
        out_specs=pl.BlockSpec((CAP_PAD, D), lambda e, k: (e, 0)),
        out_shape=jax.ShapeDtypeStruct((NROWS, D), jnp.float32),
    )(gathered, W1, W2, W3)


_PROWS = T // _NW


def _perm_body(slot_hbm, routed_hbm, out_hbm, idx_v, rows_v, sem):
    wid = lax.axis_index("s") * _NC + lax.axis_index("c")
    base = wid * _PROWS
    pltpu.sync_copy(slot_hbm.at[pl.ds(base, _PROWS)], idx_v)
    pltpu.async_copy(routed_hbm.at[idx_v], rows_v, sem).wait()
    pltpu.sync_copy(rows_v, out_hbm.at[pl.ds(base, _PROWS)])


def _perm(slot, routed):
    mesh = plsc.VectorSubcoreMesh(core_axis_name="c", subcore_axis_name="s",
                                  num_cores=_NC, num_subcores=_NS)
    return pl.kernel(
        _perm_body,
        out_type=jax.ShapeDtypeStruct((T, D), jnp.float32),
        mesh=mesh,
        scratch_types=[
            pltpu.VMEM((_PROWS,), jnp.int32),
            pltpu.VMEM((_PROWS, D), jnp.float32),
            pltpu.SemaphoreType.DMA,
        ],
        compiler_params=pltpu.CompilerParams(needs_layout_passes=False),
    )(slot, routed)


_SKC = 128
_NSKC = DSH // _SKC


def _shared_body(x_ref, s1_ref, s2_ref, s3_ref, out_ref):
    si = pl.program_id(0)
    kc = pl.program_id(1)
    xx = x_ref[...]
    h1 = jnp.dot(xx, s1_ref[0], preferred_element_type=jnp.float32)
    h2 = jnp.dot(xx, s2_ref[0], preferred_element_type=jnp.float32)
    h = h1 * (1.0 / (1.0 + jnp.exp(-h1))) * h2
    o = jnp.dot(h, s3_ref[0], preferred_element_type=jnp.float32)

    @pl.when((si == 0) & (kc == 0))
    def _init():
        out_ref[...] = o

    @pl.when((si != 0) | (kc != 0))
    def _acc():
        out_ref[...] += o


def _shared_ffn(xr, S1, S2, S3):
    return pl.pallas_call(
        _shared_body,
        grid=(NSH, _NSKC),
        in_specs=[
            pl.BlockSpec((T, D), lambda s, k: (0, 0)),
            pl.BlockSpec((1, D, _SKC), lambda s, k: (s, 0, k)),
            pl.BlockSpec((1, D, _SKC), lambda s, k: (s, 0, k)),
            pl.BlockSpec((1, _SKC, D), lambda s, k: (s, k, 0)),
        ],
        out_specs=pl.BlockSpec((T, D), lambda s, k: (0, 0)),
        out_shape=jax.ShapeDtypeStruct((T, D), jnp.float32),
    )(xr, S1, S2, S3)


_AM = 512
_NAM = T // _AM


def _assemble_body(x_ref, sh_ref, rp_ref, kept_ref, counts_ref, out_ref, aux_ref):
    k = kept_ref[...]
    xx = x_ref[...]
    out_ref[...] = sh_ref[...] + k * rp_ref[...] + (1.0 - k) * xx

    @pl.when(pl.program_id(0) == 0)
    def _aux():
        mean = jnp.float32(0.0)
        for e in range(E):
            mean += counts_ref[0, e].astype(jnp.float32)
        mean = mean / E
        var = jnp.float32(0.0)
        for e in range(E):
            d = counts_ref[0, e].astype(jnp.float32) - mean
            var += d * d
        var = var / (E - 1)
        aux_ref[...] = jnp.full((1, 1), 0.01 * E * jnp.sqrt(var), jnp.float32)


def _assemble(xr, shared_out, routed_perm, kept, counts):
    return pl.pallas_call(
        _assemble_body,
        grid=(_NAM,),
        in_specs=[
            pl.BlockSpec((_AM, D), lambda m: (m, 0)),
            pl.BlockSpec((_AM, D), lambda m: (m, 0)),
            pl.BlockSpec((_AM, D), lambda m: (m, 0)),
            pl.BlockSpec((_AM, 1), lambda m: (m, 0)),
            pl.BlockSpec(memory_space=pltpu.SMEM),
        ],
        out_specs=[
            pl.BlockSpec((_AM, D), lambda m: (m, 0)),
            pl.BlockSpec((1, 1), lambda m: (0, 0)),
        ],
        out_shape=[
            jax.ShapeDtypeStruct((T, D), jnp.float32),
            jax.ShapeDtypeStruct((1, 1), jnp.float32),
        ],
    )(xr, shared_out, routed_perm, kept, counts)


def kernel(x, gate_w, W1, W2, W3, S1, S2, S3):
    xr = x.reshape(T, D)
    eidx = _router(xr, gate_w).reshape(T)
    gathered, slot, kept, counts = _routing_gather(eidx, xr)
    routed = _expert_ffn(gathered, W1, W2, W3)
    routed_perm = _perm(slot, routed)
    shared_out = _shared_ffn(xr, S1, S2, S3)
    final, aux = _assemble(xr, shared_out, routed_perm,
                           kept.reshape(T, 1), counts.reshape(1, 16))
    return final.reshape(x.shape), aux.reshape(())


# speedup vs baseline: 1.3400x; 1.0159x over previous
import functools

import jax
import jax.numpy as jnp
from jax import lax
from jax.experimental import pallas as pl
from jax.experimental.pallas import tpu as pltpu
from jax.experimental.pallas import tpu_sc as plsc

T = 2048
D = 1024
DFF = 4096
E = 8
CAP = 307
CAP_PAD = 384
NROWS = E * CAP_PAD
NSH = 2
DSH = DFF // 2

_HI = jax.lax.Precision.HIGHEST


def _router_body(x_ref, gw_ref, out_ref):
    logits = jnp.dot(x_ref[...].astype(jnp.bfloat16),
                     gw_ref[...].astype(jnp.bfloat16),
                     preferred_element_type=jnp.float32)
    mx = jnp.max(logits, axis=1, keepdims=True)
    iot = lax.broadcasted_iota(jnp.int32, (T, E), 1)
    pick = jnp.where(logits == mx, iot, E)
    out_ref[...] = jnp.min(pick, axis=1, keepdims=True)


def _router(xr, gate_w):
    return pl.pallas_call(
        _router_body,
        out_shape=jax.ShapeDtypeStruct((T, 1), jnp.int32),
    )(xr, gate_w)


_NC = 2
_NS = 16
_NW = _NC * _NS
_GROWS = NROWS // _NW


def _routing_body(eidx_hbm, x_hbm, gathered_hbm, slot_hbm, kept_hbm, counts_hbm,
                  eidx_v, ids_v, slot_v, kept_v, cnt_v, ids_sh, idx_v, rows_v, sem):
    cid = lax.axis_index("c")
    sid = lax.axis_index("s")
    wid = sid * _NC + cid

    @pl.when(sid == 0)
    def _routing():
        pltpu.sync_copy(eidx_hbm, eidx_v)
        zero16 = jnp.zeros((16,), jnp.int32)
        for i in range(NROWS // 16):
            ids_v[pl.ds(16 * i, 16)] = zero16
        lanes = lax.iota(jnp.int32, 16)

        def body(i, carry):
            v = eidx_v[pl.ds(i * 16, 16)]
            tok = i * 16 + lanes
            slotv = jnp.zeros((16,), jnp.int32)
            keepv = jnp.zeros((16,), jnp.bool_)
            new_carry = []
            for e in range(E):
                m = v == e
                incl = plsc.cumsum(m.astype(jnp.int32))
                rank = carry[e] + incl - 1
                cnt = plsc.all_reduce_population_count(m)
                k = m & (rank < CAP)
                slotv = jnp.where(k, e * CAP_PAD + rank, slotv)
                keepv = keepv | k
                new_carry.append(carry[e] + cnt)
            slot_v[pl.ds(i * 16, 16)] = slotv
            kept_v[pl.ds(i * 16, 16)] = jnp.where(keepv, 1.0, 0.0)
            plsc.store_scatter(ids_v, [slotv], tok, mask=keepv)
            return tuple(new_carry)

        init = tuple(jnp.zeros((16,), jnp.int32) for _ in range(E))
        carry = lax.fori_loop(0, T // 16, body, init)
        cvec = jnp.zeros((16,), jnp.int32)
        for e in range(E):
            cvec = jnp.where(lanes == e, carry[e], cvec)
        cnt_v[...] = cvec
        pltpu.sync_copy(ids_v, ids_sh)

        @pl.when(cid == 0)
        def _hbm_outs():
            pltpu.sync_copy(cnt_v, counts_hbm)
            pltpu.sync_copy(slot_v, slot_hbm)
            pltpu.sync_copy(kept_v, kept_hbm)

    plsc.subcore_barrier()
    base = wid * _GROWS
    pltpu.sync_copy(ids_sh.at[pl.ds(base, _GROWS)], idx_v)
    pltpu.async_copy(x_hbm.at[idx_v], rows_v, sem).wait()
    pltpu.sync_copy(rows_v, gathered_hbm.at[pl.ds(base, _GROWS)])


def _routing_gather(eidx, xr):
    mesh = plsc.VectorSubcoreMesh(core_axis_name="c", subcore_axis_name="s",
                                  num_cores=_NC, num_subcores=_NS)
    return pl.kernel(
        _routing_body,
        out_type=[
            jax.ShapeDtypeStruct((NROWS, D), jnp.float32),
            jax.ShapeDtypeStruct((T,), jnp.int32),
            jax.ShapeDtypeStruct((T,), jnp.float32),
            jax.ShapeDtypeStruct((16,), jnp.int32),
        ],
        mesh=mesh,
        scratch_types=[
            pltpu.VMEM((T,), jnp.int32),
            pltpu.VMEM((NROWS,), jnp.int32),
            pltpu.VMEM((T,), jnp.int32),
            pltpu.VMEM((T,), jnp.float32),
            pltpu.VMEM((16,), jnp.int32),
            pltpu.VMEM_SHARED((NROWS,), jnp.int32),
            pltpu.VMEM((_GROWS,), jnp.int32),
            pltpu.VMEM((_GROWS, D), jnp.float32),
            pltpu.SemaphoreType.DMA,
        ],
        compiler_params=pltpu.CompilerParams(needs_layout_passes=False),
    )(eidx, xr)


_KC = 512
_NKC = DFF // _KC


def _ffn_body(g_ref, w1_ref, w2_ref, w3_ref, out_ref):
    g = g_ref[...].astype(jnp.bfloat16)
    h1 = jnp.dot(g, w1_ref[0].astype(jnp.bfloat16),
                 preferred_element_type=jnp.float32)
    h2 = jnp.dot(g, w2_ref[0].astype(jnp.bfloat16),
                 preferred_element_type=jnp.float32)
    h = (h1 * (1.0 / (1.0 + jnp.exp(-h1))) * h2).astype(jnp.bfloat16)
    o = jnp.dot(h, w3_ref[0].astype(jnp.bfloat16),
                preferred_element_type=jnp.float32)

    @pl.when(pl.program_id(1) == 0)
    def _init():
        out_ref[...] = o

    @pl.when(pl.program_id(1) != 0)
    def _acc():
        out_ref[...] += o


def _expert_ffn(gathered, W1, W2, W3):
    return pl.pallas_call(
        _ffn_body,
        grid=(E, _NKC),
        in_specs=[
            pl.BlockSpec((CAP_PAD, D), lambda e, k: (e, 0)),
            pl.BlockSpec((1, D, _KC), lambda e, k: (e, 0, k)),
            pl.BlockSpec((1, D, _KC), lambda e, k: (e, 0, k)),
            pl.BlockSpec((1, _KC, D), lambda e, k: (e, k, 0)),
        ],
        out_specs=pl.BlockSpec((CAP_PAD, D), lambda e, k: (e, 0)),
        out_shape=jax.ShapeDtypeStruct((NROWS, D), jnp.float32),
    )(gathered, W1, W2, W3)


_PROWS = T // _NW


def _perm_body(slot_hbm, routed_hbm, out_hbm, idx_v, rows_v, sem):
    wid = lax.axis_index("s") * _NC + lax.axis_index("c")
    base = wid * _PROWS
    pltpu.sync_copy(slot_hbm.at[pl.ds(base, _PROWS)], idx_v)
    pltpu.async_copy(routed_hbm.at[idx_v], rows_v, sem).wait()
    pltpu.sync_copy(rows_v, out_hbm.at[pl.ds(base, _PROWS)])


def _perm(slot, routed):
    mesh = plsc.VectorSubcoreMesh(core_axis_name="c", subcore_axis_name="s",
                                  num_cores=_NC, num_subcores=_NS)
    return pl.kernel(
        _perm_body,
        out_type=jax.ShapeDtypeStruct((T, D), jnp.float32),
        mesh=mesh,
        scratch_types=[
            pltpu.VMEM((_PROWS,), jnp.int32),
            pltpu.VMEM((_PROWS, D), jnp.float32),
            pltpu.SemaphoreType.DMA,
        ],
        compiler_params=pltpu.CompilerParams(needs_layout_passes=False),
    )(slot, routed)


_SKC = 128
_NSKC = DSH // _SKC


def _shared_body(x_ref, s1_ref, s2_ref, s3_ref, out_ref):
    si = pl.program_id(0)
    kc = pl.program_id(1)
    xx = x_ref[...].astype(jnp.bfloat16)
    h1 = jnp.dot(xx, s1_ref[0].astype(jnp.bfloat16),
                 preferred_element_type=jnp.float32)
    h2 = jnp.dot(xx, s2_ref[0].astype(jnp.bfloat16),
                 preferred_element_type=jnp.float32)
    h = (h1 * (1.0 / (1.0 + jnp.exp(-h1))) * h2).astype(jnp.bfloat16)
    o = jnp.dot(h, s3_ref[0].astype(jnp.bfloat16),
                preferred_element_type=jnp.float32)

    @pl.when((si == 0) & (kc == 0))
    def _init():
        out_ref[...] = o

    @pl.when((si != 0) | (kc != 0))
    def _acc():
        out_ref[...] += o


def _shared_ffn(xr, S1, S2, S3):
    return pl.pallas_call(
        _shared_body,
        grid=(NSH, _NSKC),
        in_specs=[
            pl.BlockSpec((T, D), lambda s, k: (0, 0)),
            pl.BlockSpec((1, D, _SKC), lambda s, k: (s, 0, k)),
            pl.BlockSpec((1, D, _SKC), lambda s, k: (s, 0, k)),
            pl.BlockSpec((1, _SKC, D), lambda s, k: (s, k, 0)),
        ],
        out_specs=pl.BlockSpec((T, D), lambda s, k: (0, 0)),
        out_shape=jax.ShapeDtypeStruct((T, D), jnp.float32),
    )(xr, S1, S2, S3)


_AM = 512
_NAM = T // _AM


def _assemble_body(x_ref, sh_ref, rp_ref, kept_ref, counts_ref, out_ref, aux_ref):
    k = kept_ref[...]
    xx = x_ref[...]
    out_ref[...] = sh_ref[...] + k * rp_ref[...] + (1.0 - k) * xx

    @pl.when(pl.program_id(0) == 0)
    def _aux():
        mean = jnp.float32(0.0)
        for e in range(E):
            mean += counts_ref[0, e].astype(jnp.float32)
        mean = mean / E
        var = jnp.float32(0.0)
        for e in range(E):
            d = counts_ref[0, e].astype(jnp.float32) - mean
            var += d * d
        var = var / (E - 1)
        aux_ref[...] = jnp.full((1, 1), 0.01 * E * jnp.sqrt(var), jnp.float32)


def _assemble(xr, shared_out, routed_perm, kept, counts):
    return pl.pallas_call(
        _assemble_body,
        grid=(_NAM,),
        in_specs=[
            pl.BlockSpec((_AM, D), lambda m: (m, 0)),
            pl.BlockSpec((_AM, D), lambda m: (m, 0)),
            pl.BlockSpec((_AM, D), lambda m: (m, 0)),
            pl.BlockSpec((_AM, 1), lambda m: (m, 0)),
            pl.BlockSpec(memory_space=pltpu.SMEM),
        ],
        out_specs=[
            pl.BlockSpec((_AM, D), lambda m: (m, 0)),
            pl.BlockSpec((1, 1), lambda m: (0, 0)),
        ],
        out_shape=[
            jax.ShapeDtypeStruct((T, D), jnp.float32),
            jax.ShapeDtypeStruct((1, 1), jnp.float32),
        ],
    )(xr, shared_out, routed_perm, kept, counts)


def kernel(x, gate_w, W1, W2, W3, S1, S2, S3):
    xr = x.reshape(T, D)
    eidx = _router(xr, gate_w).reshape(T)
    gathered, slot, kept, counts = _routing_gather(eidx, xr)
    routed = _expert_ffn(gathered, W1, W2, W3)
    routed_perm = _perm(slot, routed)
    shared_out = _shared_ffn(xr, S1, S2, S3)
    final, aux = _assemble(xr, shared_out, routed_perm,
                           kept.reshape(T, 1), counts.reshape(1, 16))
    return final.reshape(x.shape), aux.reshape(())


# speedup vs baseline: 1.7041x; 1.2718x over previous
import functools

import jax
import jax.numpy as jnp
from jax import lax
from jax.experimental import pallas as pl
from jax.experimental.pallas import tpu as pltpu
from jax.experimental.pallas import tpu_sc as plsc

T = 2048
D = 1024
DFF = 4096
E = 8
CAP = 307
CAP_PAD = 384
NROWS = E * CAP_PAD
NSH = 2
DSH = DFF // 2

_HI = jax.lax.Precision.HIGHEST


def _router_body(x_ref, gw_ref, out_ref):
    logits = jnp.dot(x_ref[...].astype(jnp.bfloat16),
                     gw_ref[...].astype(jnp.bfloat16),
                     preferred_element_type=jnp.float32)
    mx = jnp.max(logits, axis=1, keepdims=True)
    iot = lax.broadcasted_iota(jnp.int32, (T, E), 1)
    pick = jnp.where(logits == mx, iot, E)
    out_ref[...] = jnp.min(pick, axis=1, keepdims=True)


def _router(xr, gate_w):
    return pl.pallas_call(
        _router_body,
        out_shape=jax.ShapeDtypeStruct((T, 1), jnp.int32),
    )(xr, gate_w)


_NC = 2
_NS = 16
_NW = _NC * _NS
_GROWS = NROWS // _NW


def _routing_body(eidx_hbm, x_hbm, gathered_hbm, slot_hbm, kept_hbm, counts_hbm,
                  eidx_v, ids_v, slot_v, kept_v, cnt_v, ids_sh, idx_v, rows_v, sem):
    cid = lax.axis_index("c")
    sid = lax.axis_index("s")
    wid = sid * _NC + cid

    @pl.when(sid == 0)
    def _routing():
        pltpu.sync_copy(eidx_hbm, eidx_v)
        zero16 = jnp.zeros((16,), jnp.int32)
        for i in range(NROWS // 16):
            ids_v[pl.ds(16 * i, 16)] = zero16
        lanes = lax.iota(jnp.int32, 16)

        def body(i, carry):
            v = eidx_v[pl.ds(i * 16, 16)]
            tok = i * 16 + lanes
            slotv = jnp.zeros((16,), jnp.int32)
            keepv = jnp.zeros((16,), jnp.bool_)
            new_carry = []
            for e in range(E):
                m = v == e
                incl = plsc.cumsum(m.astype(jnp.int32))
                rank = carry[e] + incl - 1
                cnt = plsc.all_reduce_population_count(m)
                k = m & (rank < CAP)
                slotv = jnp.where(k, e * CAP_PAD + rank, slotv)
                keepv = keepv | k
                new_carry.append(carry[e] + cnt)
            slot_v[pl.ds(i * 16, 16)] = slotv
            kept_v[pl.ds(i * 16, 16)] = jnp.where(keepv, 1.0, 0.0)
            plsc.store_scatter(ids_v, [slotv], tok, mask=keepv)
            return tuple(new_carry)

        init = tuple(jnp.zeros((16,), jnp.int32) for _ in range(E))
        carry = lax.fori_loop(0, T // 16, body, init)
        cvec = jnp.zeros((16,), jnp.int32)
        for e in range(E):
            cvec = jnp.where(lanes == e, carry[e], cvec)
        cnt_v[...] = cvec
        pltpu.sync_copy(ids_v, ids_sh)

        @pl.when(cid == 0)
        def _hbm_outs():
            pltpu.sync_copy(cnt_v, counts_hbm)
            pltpu.sync_copy(slot_v, slot_hbm)
            pltpu.sync_copy(kept_v, kept_hbm)

    plsc.subcore_barrier()
    base = wid * _GROWS
    pltpu.sync_copy(ids_sh.at[pl.ds(base, _GROWS)], idx_v)
    pltpu.async_copy(x_hbm.at[idx_v], rows_v, sem).wait()
    pltpu.sync_copy(rows_v, gathered_hbm.at[pl.ds(base, _GROWS)])


def _routing_gather(eidx, xr):
    mesh = plsc.VectorSubcoreMesh(core_axis_name="c", subcore_axis_name="s",
                                  num_cores=_NC, num_subcores=_NS)
    return pl.kernel(
        _routing_body,
        out_type=[
            jax.ShapeDtypeStruct((NROWS, D), jnp.float32),
            jax.ShapeDtypeStruct((T,), jnp.int32),
            jax.ShapeDtypeStruct((T,), jnp.float32),
            jax.ShapeDtypeStruct((16,), jnp.int32),
        ],
        mesh=mesh,
        scratch_types=[
            pltpu.VMEM((T,), jnp.int32),
            pltpu.VMEM((NROWS,), jnp.int32),
            pltpu.VMEM((T,), jnp.int32),
            pltpu.VMEM((T,), jnp.float32),
            pltpu.VMEM((16,), jnp.int32),
            pltpu.VMEM_SHARED((NROWS,), jnp.int32),
            pltpu.VMEM((_GROWS,), jnp.int32),
            pltpu.VMEM((_GROWS, D), jnp.float32),
            pltpu.SemaphoreType.DMA,
        ],
        compiler_params=pltpu.CompilerParams(needs_layout_passes=False),
    )(eidx, xr)


_KC = 1024
_NKC = DFF // _KC


def _ffn_body(g_ref, w1_ref, w2_ref, w3_ref, out_ref):
    g = g_ref[...].astype(jnp.bfloat16)
    h1 = jnp.dot(g, w1_ref[0].astype(jnp.bfloat16),
                 preferred_element_type=jnp.float32)
    h2 = jnp.dot(g, w2_ref[0].astype(jnp.bfloat16),
                 preferred_element_type=jnp.float32)
    h = (h1 * (1.0 / (1.0 + jnp.exp(-h1))) * h2).astype(jnp.bfloat16)
    o = jnp.dot(h, w3_ref[0].astype(jnp.bfloat16),
                preferred_element_type=jnp.float32)

    @pl.when(pl.program_id(1) == 0)
    def _init():
        out_ref[...] = o

    @pl.when(pl.program_id(1) != 0)
    def _acc():
        out_ref[...] += o


def _expert_ffn(gathered, W1, W2, W3):
    return pl.pallas_call(
        _ffn_body,
        grid=(E, _NKC),
        in_specs=[
            pl.BlockSpec((CAP_PAD, D), lambda e, k: (e, 0)),
            pl.BlockSpec((1, D, _KC), lambda e, k: (e, 0, k)),
            pl.BlockSpec((1, D, _KC), lambda e, k: (e, 0, k)),
            pl.BlockSpec((1, _KC, D), lambda e, k: (e, k, 0)),
        ],
        out_specs=pl.BlockSpec((CAP_PAD, D), lambda e, k: (e, 0)),
        out_shape=jax.ShapeDtypeStruct((NROWS, D), jnp.float32),
    )(gathered, W1, W2, W3)


_PROWS = T // _NW


def _perm_body(slot_hbm, routed_hbm, out_hbm, idx_v, rows_v, sem):
    wid = lax.axis_index("s") * _NC + lax.axis_index("c")
    base = wid * _PROWS
    pltpu.sync_copy(slot_hbm.at[pl.ds(base, _PROWS)], idx_v)
    pltpu.async_copy(routed_hbm.at[idx_v], rows_v, sem).wait()
    pltpu.sync_copy(rows_v, out_hbm.at[pl.ds(base, _PROWS)])


def _perm(slot, routed):
    mesh = plsc.VectorSubcoreMesh(core_axis_name="c", subcore_axis_name="s",
                                  num_cores=_NC, num_subcores=_NS)
    return pl.kernel(
        _perm_body,
        out_type=jax.ShapeDtypeStruct((T, D), jnp.float32),
        mesh=mesh,
        scratch_types=[
            pltpu.VMEM((_PROWS,), jnp.int32),
            pltpu.VMEM((_PROWS, D), jnp.float32),
            pltpu.SemaphoreType.DMA,
        ],
        compiler_params=pltpu.CompilerParams(needs_layout_passes=False),
    )(slot, routed)


_SKC = 512
_NSKC = DSH // _SKC


def _shared_body(x_ref, s1_ref, s2_ref, s3_ref, out_ref):
    si = pl.program_id(0)
    kc = pl.program_id(1)
    xx = x_ref[...].astype(jnp.bfloat16)
    h1 = jnp.dot(xx, s1_ref[0].astype(jnp.bfloat16),
                 preferred_element_type=jnp.float32)
    h2 = jnp.dot(xx, s2_ref[0].astype(jnp.bfloat16),
                 preferred_element_type=jnp.float32)
    h = (h1 * (1.0 / (1.0 + jnp.exp(-h1))) * h2).astype(jnp.bfloat16)
    o = jnp.dot(h, s3_ref[0].astype(jnp.bfloat16),
                preferred_element_type=jnp.float32)

    @pl.when((si == 0) & (kc == 0))
    def _init():
        out_ref[...] = o

    @pl.when((si != 0) | (kc != 0))
    def _acc():
        out_ref[...] += o


def _shared_ffn(xr, S1, S2, S3):
    return pl.pallas_call(
        _shared_body,
        grid=(NSH, _NSKC),
        in_specs=[
            pl.BlockSpec((T, D), lambda s, k: (0, 0)),
            pl.BlockSpec((1, D, _SKC), lambda s, k: (s, 0, k)),
            pl.BlockSpec((1, D, _SKC), lambda s, k: (s, 0, k)),
            pl.BlockSpec((1, _SKC, D), lambda s, k: (s, k, 0)),
        ],
        out_specs=pl.BlockSpec((T, D), lambda s, k: (0, 0)),
        out_shape=jax.ShapeDtypeStruct((T, D), jnp.float32),
    )(xr, S1, S2, S3)


_AM = 512
_NAM = T // _AM


def _assemble_body(x_ref, sh_ref, rp_ref, kept_ref, counts_ref, out_ref, aux_ref):
    k = kept_ref[...]
    xx = x_ref[...]
    out_ref[...] = sh_ref[...] + k * rp_ref[...] + (1.0 - k) * xx

    @pl.when(pl.program_id(0) == 0)
    def _aux():
        mean = jnp.float32(0.0)
        for e in range(E):
            mean += counts_ref[0, e].astype(jnp.float32)
        mean = mean / E
        var = jnp.float32(0.0)
        for e in range(E):
            d = counts_ref[0, e].astype(jnp.float32) - mean
            var += d * d
        var = var / (E - 1)
        aux_ref[...] = jnp.full((1, 1), 0.01 * E * jnp.sqrt(var), jnp.float32)


def _assemble(xr, shared_out, routed_perm, kept, counts):
    return pl.pallas_call(
        _assemble_body,
        grid=(_NAM,),
        in_specs=[
            pl.BlockSpec((_AM, D), lambda m: (m, 0)),
            pl.BlockSpec((_AM, D), lambda m: (m, 0)),
            pl.BlockSpec((_AM, D), lambda m: (m, 0)),
            pl.BlockSpec((_AM, 1), lambda m: (m, 0)),
            pl.BlockSpec(memory_space=pltpu.SMEM),
        ],
        out_specs=[
            pl.BlockSpec((_AM, D), lambda m: (m, 0)),
            pl.BlockSpec((1, 1), lambda m: (0, 0)),
        ],
        out_shape=[
            jax.ShapeDtypeStruct((T, D), jnp.float32),
            jax.ShapeDtypeStruct((1, 1), jnp.float32),
        ],
    )(xr, shared_out, routed_perm, kept, counts)


def kernel(x, gate_w, W1, W2, W3, S1, S2, S3):
    xr = x.reshape(T, D)
    eidx = _router(xr, gate_w).reshape(T)
    shared_out = _shared_ffn(xr, S1, S2, S3)
    gathered, slot, kept, counts = _routing_gather(eidx, xr)
    routed = _expert_ffn(gathered, W1, W2, W3)
    routed_perm = _perm(slot, routed)
    final, aux = _assemble(xr, shared_out, routed_perm,
                           kept.reshape(T, 1), counts.reshape(1, 16))
    return final.reshape(x.shape), aux.reshape(())
